# baseline (speedup 1.0000x reference)
import math

import jax
import jax.numpy as jnp
from jax.experimental import pallas as pl
from jax.experimental.pallas import tpu as pltpu

B, N = 32, 64
DN, DE, DG = 64, 32, 16
OUT_E = 32
N_HEADS = 4
HEAD_DIM = OUT_E // N_HEADS
HDDN = 64
EDGE_IN = DE + 2 * DN
H2 = 2 * HDDN
INV_SQRT_OUT = 1.0 / math.sqrt(OUT_E)


def _edge_block_kernel(nodes_ref, edges_ref, globs_ref,
                       Ws_ref, Wr_ref, We_ref, Wc_ref,
                       gw_ref, cvec0_ref, W2B_ref, b2B_ref,
                       feats_ref, pooled_ref):
    nodes = nodes_ref[0]
    edges = edges_ref[...]
    glb = globs_ref[0]
    edges3 = edges.reshape(N, N, DE)

    nsum = jnp.sum(nodes, axis=1)
    nsq = jnp.sum(nodes * nodes, axis=1)
    es = jnp.sum(edges3, axis=2, keepdims=True)
    ess = jnp.sum(edges3 * edges3, axis=2, keepdims=True)
    mu = (nsum.reshape(N, 1, 1) + nsum.reshape(1, N, 1) + es) * (1.0 / EDGE_IN)
    msq = (nsq.reshape(N, 1, 1) + nsq.reshape(1, N, 1) + ess) * (1.0 / EDGE_IN)
    inv = jax.lax.rsqrt(msq - mu * mu + 1e-5)

    S = nodes @ Ws_ref[...]
    R = nodes @ Wr_ref[...]
    Et = (edges @ We_ref[...]).reshape(N, N, H2)
    cvec = (glb @ Wc_ref[...] + cvec0_ref[...]).reshape(1, 1, H2)
    gw = gw_ref[...].reshape(1, 1, H2)

    A = S.reshape(N, 1, H2) + R.reshape(1, N, H2) + Et
    h = (A - mu * gw) * inv + cvec
    h = h * jax.nn.sigmoid(h)

    out = (h.reshape(N * N, H2) @ W2B_ref[...] + b2B_ref[...]).reshape(N, N, 2 * OUT_E)
    feats = out[:, :, :OUT_E] + edges3
    aw = out[:, :, OUT_E:]

    m = jnp.max(aw, axis=0, keepdims=True)
    ex = jnp.exp(aw - m)
    w = ex / jnp.sum(ex, axis=0, keepdims=True)
    pooled = jnp.sum(feats * w, axis=0) * INV_SQRT_OUT

    feats_ref[...] = feats.reshape(N * N, OUT_E)
    pooled_ref[0] = pooled


def kernel(nodes, edges, globs, adjmat, pre_ln_g, pre_ln_b,
           feat_W1, feat_b1, feat_W2, feat_b2,
           attn_W1, attn_b1, attn_W2, attn_b2):
    del adjmat
    globs3 = globs.reshape(B, 1, DG)

    W1B = jnp.concatenate([feat_W1, attn_W1], axis=1)
    W1Bg = W1B[:EDGE_IN] * pre_ln_g[:, None]
    Ws = W1Bg[:DN]
    Wr = W1Bg[DN:2 * DN]
    We = W1Bg[2 * DN:]
    Wc = W1B[EDGE_IN:]
    gw = (pre_ln_g @ W1B[:EDGE_IN]).reshape(1, H2)
    b1B = jnp.concatenate([feat_b1, attn_b1])
    cvec0 = (pre_ln_b @ W1B[:EDGE_IN] + b1B).reshape(1, H2)
    aW2r = jnp.repeat(attn_W2, HEAD_DIM, axis=1)
    ab2r = jnp.repeat(attn_b2, HEAD_DIM)
    z = jnp.zeros((HDDN, OUT_E), jnp.float32)
    W2B = jnp.concatenate([
        jnp.concatenate([feat_W2, z], axis=1),
        jnp.concatenate([z, aW2r], axis=1),
    ], axis=0)
    b2B = jnp.concatenate([feat_b2, ab2r]).reshape(1, 2 * OUT_E)

    full = lambda shape: pl.BlockSpec(shape, lambda b: tuple(0 for _ in shape))
    feats, pooled = pl.pallas_call(
        _edge_block_kernel,
        grid=(B,),
        in_specs=[
            pl.BlockSpec((1, N, DN), lambda b: (b, 0, 0)),
            pl.BlockSpec((N * N, DE), lambda b: (b, 0)),
            pl.BlockSpec((1, 1, DG), lambda b: (b, 0, 0)),
            full((DN, H2)),
            full((DN, H2)),
            full((DE, H2)),
            full((DG, H2)),
            full((1, H2)),
            full((1, H2)),
            full((H2, 2 * OUT_E)),
            full((1, 2 * OUT_E)),
        ],
        out_specs=[
            pl.BlockSpec((N * N, OUT_E), lambda b: (b, 0)),
            pl.BlockSpec((1, N, OUT_E), lambda b: (b, 0, 0)),
        ],
        out_shape=[
            jax.ShapeDtypeStruct((B * N * N, OUT_E), jnp.float32),
            jax.ShapeDtypeStruct((B, N, OUT_E), jnp.float32),
        ],
        compiler_params=pltpu.CompilerParams(
            dimension_semantics=("parallel",),
        ),
    )(nodes, edges, globs3, Ws, Wr, We, Wc, gw, cvec0, W2B, b2B)
    return feats, pooled


# speedup vs baseline: 2.0258x; 2.0258x over previous
import math

import jax
import jax.numpy as jnp
from jax.experimental import pallas as pl
from jax.experimental.pallas import tpu as pltpu

B, N = 32, 64
DN, DE, DG = 64, 32, 16
OUT_E = 32
N_HEADS = 4
HEAD_DIM = OUT_E // N_HEADS
HDDN = 64
EDGE_IN = DE + 2 * DN
H2 = 2 * HDDN
E1 = N * N
INV_SQRT_OUT = 1.0 / math.sqrt(OUT_E)


def _edge_block_kernel(nodes_ref, edgesT_ref, globsT_ref,
                       Ws_ref, Wr_ref, WeT_ref, WcT_ref,
                       gwc_ref, cvec0_ref, W2BT_ref, b2c_ref,
                       selT_ref, selP_ref, onesDE_ref,
                       featsT_ref, pooledT_ref):
    nodes = nodes_ref[0]
    eT = edgesT_ref[...]
    glbT = globsT_ref[0]

    S = nodes @ Ws_ref[...]
    R = nodes @ Wr_ref[...]
    nsum = jnp.sum(nodes, axis=1, keepdims=True)
    nsq = jnp.sum(nodes * nodes, axis=1, keepdims=True)
    SRT = jnp.concatenate([S, R], axis=0).T
    nsrow = jnp.concatenate([nsum, nsum], axis=0).T
    nqrow = jnp.concatenate([nsq, nsq], axis=0).T

    AT = SRT @ selT_ref[...] + WeT_ref[...] @ eT
    nsb = nsrow @ selT_ref[...]
    nqb = nqrow @ selT_ref[...]

    esT = onesDE_ref[...] @ eT
    essT = onesDE_ref[...] @ (eT * eT)
    t = (nsb + esT) * (1.0 / EDGE_IN)
    msq = (nqb + essT) * (1.0 / EDGE_IN)
    inv = jax.lax.rsqrt(msq - t * t + 1e-5)
    u = t * inv

    cvec = WcT_ref[...] @ glbT + cvec0_ref[...]
    h = AT * inv - gwc_ref[...] * u + cvec
    h = h * jax.nn.sigmoid(h)

    out = W2BT_ref[...] @ h + b2c_ref[...]
    featsT = out[:OUT_E] + eT
    ex = jnp.exp(out[OUT_E:])

    num = (featsT * ex) @ selP_ref[...]
    den = ex @ selP_ref[...]
    pooledT = num / den * INV_SQRT_OUT

    featsT_ref[...] = featsT
    pooledT_ref[0] = pooledT


def kernel(nodes, edges, globs, adjmat, pre_ln_g, pre_ln_b,
           feat_W1, feat_b1, feat_W2, feat_b2,
           attn_W1, attn_b1, attn_W2, attn_b2):
    del adjmat
    edgesT = edges.T
    globsT = globs.reshape(B, DG, 1)

    W1B = jnp.concatenate([feat_W1, attn_W1], axis=1)
    W1Bg = W1B[:EDGE_IN] * pre_ln_g[:, None]
    Ws = W1Bg[:DN]
    Wr = W1Bg[DN:2 * DN]
    WeT = W1Bg[2 * DN:].T
    WcT = W1B[EDGE_IN:].T
    gwc = (pre_ln_g @ W1B[:EDGE_IN]).reshape(H2, 1)
    b1B = jnp.concatenate([feat_b1, attn_b1])
    cvec0 = (pre_ln_b @ W1B[:EDGE_IN] + b1B).reshape(H2, 1)
    aW2r = jnp.repeat(attn_W2, HEAD_DIM, axis=1)
    ab2r = jnp.repeat(attn_b2, HEAD_DIM)
    z = jnp.zeros((HDDN, OUT_E), jnp.float32)
    W2BT = jnp.concatenate([
        jnp.concatenate([feat_W2, z], axis=1),
        jnp.concatenate([z, aW2r], axis=1),
    ], axis=0).T
    b2c = jnp.concatenate([feat_b2, ab2r]).reshape(2 * OUT_E, 1)

    eidx = jnp.arange(E1, dtype=jnp.int32)
    nidx = jnp.arange(N, dtype=jnp.int32)
    sel_s = (eidx[None, :] // N == nidx[:, None]).astype(jnp.float32)
    sel_r = (eidx[None, :] % N == nidx[:, None]).astype(jnp.float32)
    selT = jnp.concatenate([sel_s, sel_r], axis=0)
    selP = sel_r.T
    onesDE = jnp.ones((1, DE), jnp.float32)

    full = lambda shape: pl.BlockSpec(shape, lambda b: tuple(0 for _ in shape))
    featsT, pooledT = pl.pallas_call(
        _edge_block_kernel,
        grid=(B,),
        in_specs=[
            pl.BlockSpec((1, N, DN), lambda b: (b, 0, 0)),
            pl.BlockSpec((DE, E1), lambda b: (0, b)),
            pl.BlockSpec((1, DG, 1), lambda b: (b, 0, 0)),
            full((DN, H2)),
            full((DN, H2)),
            full((H2, DE)),
            full((H2, DG)),
            full((H2, 1)),
            full((H2, 1)),
            full((2 * OUT_E, H2)),
            full((2 * OUT_E, 1)),
            full((2 * N, E1)),
            full((E1, N)),
            full((1, DE)),
        ],
        out_specs=[
            pl.BlockSpec((OUT_E, E1), lambda b: (0, b)),
            pl.BlockSpec((1, OUT_E, N), lambda b: (b, 0, 0)),
        ],
        out_shape=[
            jax.ShapeDtypeStruct((OUT_E, B * E1), jnp.float32),
            jax.ShapeDtypeStruct((B, OUT_E, N), jnp.float32),
        ],
        compiler_params=pltpu.CompilerParams(
            dimension_semantics=("parallel",),
        ),
    )(nodes, edgesT, globsT, Ws, Wr, WeT, WcT, gwc, cvec0, W2BT, b2c,
      selT, selP, onesDE)
    feats = featsT.T
    pooled = pooledT.transpose(0, 2, 1)
    return feats, pooled


# speedup vs baseline: 2.2423x; 1.1069x over previous
import math

import jax
import jax.numpy as jnp
import numpy as np
from jax.experimental import pallas as pl
from jax.experimental.pallas import tpu as pltpu

B, N = 32, 64
DN, DE, DG = 64, 32, 16
OUT_E = 32
N_HEADS = 4
HEAD_DIM = OUT_E // N_HEADS
HDDN = 64
EDGE_IN = DE + 2 * DN
H2 = 2 * HDDN
E1 = N * N
BP = 2
EW = BP * E1
INV_SQRT_OUT = 1.0 / math.sqrt(OUT_E)


def _edge_block_kernel(nodes_ref, edgesT_ref, globsT_ref,
                       Ws_ref, Wr_ref, WeT_ref, WcT_ref,
                       gwc_ref, cvec0_ref, W2BT_ref, b2c_ref,
                       selT_ref, selP_ref, ind_ref, onesDE_ref,
                       featsT_ref, pooledT_ref):
    nodes2 = nodes_ref[...].reshape(BP * N, DN)
    eT = edgesT_ref[...]
    glb2 = globsT_ref[...].reshape(BP, DG).T

    S = nodes2 @ Ws_ref[...]
    R = nodes2 @ Wr_ref[...]
    nsum = jnp.sum(nodes2, axis=1, keepdims=True)
    nsq = jnp.sum(nodes2 * nodes2, axis=1, keepdims=True)
    SRT = jnp.concatenate([S[:N], R[:N], S[N:], R[N:]], axis=0).T
    nsrow = jnp.concatenate([nsum[:N], nsum[:N], nsum[N:], nsum[N:]], axis=0).T
    nqrow = jnp.concatenate([nsq[:N], nsq[:N], nsq[N:], nsq[N:]], axis=0).T
    SRX = jnp.concatenate([SRT, nsrow, nqrow], axis=0)

    AX = SRX @ selT_ref[...]
    AT = AX[:H2] + WeT_ref[...] @ eT
    nsb = AX[H2:H2 + 1]
    nqb = AX[H2 + 1:H2 + 2]

    esT = onesDE_ref[...] @ eT
    essT = onesDE_ref[...] @ (eT * eT)
    t = (nsb + esT) * (1.0 / EDGE_IN)
    msq = (nqb + essT) * (1.0 / EDGE_IN)
    inv = jax.lax.rsqrt(msq - t * t + 1e-5)
    u = t * inv

    cvec2 = WcT_ref[...] @ glb2 + cvec0_ref[...]
    lhs = jnp.concatenate([-gwc_ref[...], cvec2], axis=1)
    rhs = jnp.concatenate([u, ind_ref[...]], axis=0)
    C = lhs @ rhs
    h = AT * inv + C
    h = h * jax.nn.sigmoid(h)

    out = W2BT_ref[...] @ h + b2c_ref[...]
    featsT = out[:OUT_E] + eT
    ex = jnp.exp(out[OUT_E:])

    w = featsT * ex
    for b in range(BP):
        sl = slice(b * E1, (b + 1) * E1)
        num = w[:, sl] @ selP_ref[...]
        den = ex[:, sl] @ selP_ref[...]
        pooledT_ref[b] = num / den * INV_SQRT_OUT

    featsT_ref[...] = featsT


def kernel(nodes, edges, globs, adjmat, pre_ln_g, pre_ln_b,
           feat_W1, feat_b1, feat_W2, feat_b2,
           attn_W1, attn_b1, attn_W2, attn_b2):
    del adjmat
    edgesT = edges.T
    globsT = globs.reshape(B, DG, 1)

    W1B = jnp.concatenate([feat_W1, attn_W1], axis=1)
    W1Bg = W1B[:EDGE_IN] * pre_ln_g[:, None]
    Ws = W1Bg[:DN]
    Wr = W1Bg[DN:2 * DN]
    WeT = W1Bg[2 * DN:].T
    WcT = W1B[EDGE_IN:].T
    gwc = (pre_ln_g @ W1B[:EDGE_IN]).reshape(H2, 1)
    b1B = jnp.concatenate([feat_b1, attn_b1])
    cvec0 = (pre_ln_b @ W1B[:EDGE_IN] + b1B).reshape(H2, 1)
    aW2r = jnp.repeat(attn_W2, HEAD_DIM, axis=1)
    ab2r = jnp.repeat(attn_b2, HEAD_DIM)
    z = jnp.zeros((HDDN, OUT_E), jnp.float32)
    W2BT = jnp.concatenate([
        jnp.concatenate([feat_W2, z], axis=1),
        jnp.concatenate([z, aW2r], axis=1),
    ], axis=0).T
    b2c = jnp.concatenate([feat_b2, ab2r]).reshape(2 * OUT_E, 1)

    eidx = np.arange(E1)
    nidx = np.arange(N)
    sel_s = (eidx[None, :] // N == nidx[:, None]).astype(np.float32)
    sel_r = (eidx[None, :] % N == nidx[:, None]).astype(np.float32)
    sel1 = np.concatenate([sel_s, sel_r], axis=0)
    selT = jnp.asarray(np.kron(np.eye(BP, dtype=np.float32), sel1))
    selP = jnp.asarray(sel_r.T.copy())
    ind = jnp.asarray(
        np.kron(np.eye(BP, dtype=np.float32), np.ones((1, E1), np.float32)))
    onesDE = jnp.asarray(np.ones((1, DE), np.float32))

    full = lambda shape: pl.BlockSpec(shape, lambda b: tuple(0 for _ in shape))
    featsT, pooledT = pl.pallas_call(
        _edge_block_kernel,
        grid=(B // BP,),
        in_specs=[
            pl.BlockSpec((BP, N, DN), lambda b: (b, 0, 0)),
            pl.BlockSpec((DE, EW), lambda b: (0, b)),
            pl.BlockSpec((BP, DG, 1), lambda b: (b, 0, 0)),
            full((DN, H2)),
            full((DN, H2)),
            full((H2, DE)),
            full((H2, DG)),
            full((H2, 1)),
            full((H2, 1)),
            full((2 * OUT_E, H2)),
            full((2 * OUT_E, 1)),
            full((2 * N * BP, EW)),
            full((E1, N)),
            full((BP, EW)),
            full((1, DE)),
        ],
        out_specs=[
            pl.BlockSpec((OUT_E, EW), lambda b: (0, b)),
            pl.BlockSpec((BP, OUT_E, N), lambda b: (b, 0, 0)),
        ],
        out_shape=[
            jax.ShapeDtypeStruct((OUT_E, B * E1), jnp.float32),
            jax.ShapeDtypeStruct((B, OUT_E, N), jnp.float32),
        ],
        compiler_params=pltpu.CompilerParams(
            dimension_semantics=("parallel",),
        ),
    )(nodes, edgesT, globsT, Ws, Wr, WeT, WcT, gwc, cvec0, W2BT, b2c,
      selT, selP, ind, onesDE)
    feats = featsT.T
    pooled = pooledT.transpose(0, 2, 1)
    return feats, pooled


# speedup vs baseline: 2.3356x; 1.0416x over previous
import math

import jax
import jax.numpy as jnp
import numpy as np
from jax.experimental import pallas as pl
from jax.experimental.pallas import tpu as pltpu

B, N = 32, 64
DN, DE, DG = 64, 32, 16
OUT_E = 32
N_HEADS = 4
HEAD_DIM = OUT_E // N_HEADS
HDDN = 64
EDGE_IN = DE + 2 * DN
H2 = 2 * HDDN
E1 = N * N
BP = 2
EW = BP * E1
INV_SQRT_OUT = 1.0 / math.sqrt(OUT_E)


def _prep_kernel(plg_ref, plb_ref, fW1_ref, fb1_ref, fW2_ref, fb2_ref,
                 aW1_ref, ab1_ref, aW2_ref, ab2_ref, rep_ref,
                 Ws_ref, Wr_ref, WeT_ref, WcT_ref, gwc_ref, cvec0_ref,
                 W2BT_ref, b2c_ref):
    g = plg_ref[...]
    W1B = jnp.concatenate([fW1_ref[...], aW1_ref[...]], axis=1)
    W1Bg = W1B[:EDGE_IN] * g.T
    Ws_ref[...] = W1Bg[:DN]
    Wr_ref[...] = W1Bg[DN:2 * DN]
    WeT_ref[...] = W1Bg[2 * DN:].T
    WcT_ref[...] = W1B[EDGE_IN:].T
    gwc_ref[...] = (g @ W1B[:EDGE_IN]).T
    b1B = jnp.concatenate([fb1_ref[...], ab1_ref[...]], axis=1)
    cvec0_ref[...] = (plb_ref[...] @ W1B[:EDGE_IN] + b1B).T
    aW2rT = (aW2_ref[...] @ rep_ref[...]).T
    z = jnp.zeros((OUT_E, HDDN), jnp.float32)
    W2BT_ref[...] = jnp.concatenate([
        jnp.concatenate([fW2_ref[...].T, z], axis=1),
        jnp.concatenate([z, aW2rT], axis=1),
    ], axis=0)
    b2c_ref[...] = jnp.concatenate(
        [fb2_ref[...], ab2_ref[...] @ rep_ref[...]], axis=1).T


def _edge_block_kernel(nodes_ref, edgesT_ref, globsT_ref,
                       Ws_ref, Wr_ref, WeT_ref, WcT_ref,
                       gwc_ref, cvec0_ref, W2BT_ref, b2c_ref,
                       selT_ref, selP_ref, ind_ref, onesDE_ref,
                       featsT_ref, pooledT_ref):
    nodes2 = nodes_ref[...].reshape(BP * N, DN)
    eT = edgesT_ref[...]
    glb2 = globsT_ref[...].reshape(BP, DG).T

    S = nodes2 @ Ws_ref[...]
    R = nodes2 @ Wr_ref[...]
    nsum = jnp.sum(nodes2, axis=1, keepdims=True)
    nsq = jnp.sum(nodes2 * nodes2, axis=1, keepdims=True)
    SRT = jnp.concatenate([S[:N], R[:N], S[N:], R[N:]], axis=0).T
    nsrow = jnp.concatenate([nsum[:N], nsum[:N], nsum[N:], nsum[N:]], axis=0).T
    nqrow = jnp.concatenate([nsq[:N], nsq[:N], nsq[N:], nsq[N:]], axis=0).T
    SRX = jnp.concatenate([SRT, nsrow, nqrow], axis=0)

    AX = SRX @ selT_ref[...]
    AT = AX[:H2] + WeT_ref[...] @ eT
    nsb = AX[H2:H2 + 1]
    nqb = AX[H2 + 1:H2 + 2]

    esT = onesDE_ref[...] @ eT
    essT = onesDE_ref[...] @ (eT * eT)
    t = (nsb + esT) * (1.0 / EDGE_IN)
    msq = (nqb + essT) * (1.0 / EDGE_IN)
    inv = jax.lax.rsqrt(msq - t * t + 1e-5)
    u = t * inv

    cvec2 = WcT_ref[...] @ glb2 + cvec0_ref[...]
    lhs = jnp.concatenate([-gwc_ref[...], cvec2], axis=1)
    rhs = jnp.concatenate([u, ind_ref[...]], axis=0)
    C = lhs @ rhs
    h = AT * inv + C
    h = h * jax.nn.sigmoid(h)

    out = W2BT_ref[...] @ h + b2c_ref[...]
    featsT = out[:OUT_E] + eT
    ex = jnp.exp(out[OUT_E:])

    w = featsT * ex
    for b in range(BP):
        sl = slice(b * E1, (b + 1) * E1)
        num = w[:, sl] @ selP_ref[...]
        den = ex[:, sl] @ selP_ref[...]
        pooledT_ref[b] = num / den * INV_SQRT_OUT

    featsT_ref[...] = featsT


def kernel(nodes, edges, globs, adjmat, pre_ln_g, pre_ln_b,
           feat_W1, feat_b1, feat_W2, feat_b2,
           attn_W1, attn_b1, attn_W2, attn_b2):
    del adjmat
    edgesT = edges.T
    globsT = globs.reshape(B, DG, 1)

    eidx = np.arange(E1)
    nidx = np.arange(N)
    sel_s = (eidx[None, :] // N == nidx[:, None]).astype(np.float32)
    sel_r = (eidx[None, :] % N == nidx[:, None]).astype(np.float32)
    sel1 = np.concatenate([sel_s, sel_r], axis=0)
    selT = jnp.asarray(np.kron(np.eye(BP, dtype=np.float32), sel1))
    selP = jnp.asarray(sel_r.T.copy())
    ind = jnp.asarray(
        np.kron(np.eye(BP, dtype=np.float32), np.ones((1, E1), np.float32)))
    onesDE = jnp.asarray(np.ones((1, DE), np.float32))
    rep = jnp.asarray(
        np.kron(np.eye(N_HEADS, dtype=np.float32), np.ones((1, HEAD_DIM), np.float32)))

    full = lambda shape: pl.BlockSpec(shape, lambda *_: tuple(0 for _ in shape))
    prep_out = pl.pallas_call(
        _prep_kernel,
        out_shape=[
            jax.ShapeDtypeStruct((DN, H2), jnp.float32),
            jax.ShapeDtypeStruct((DN, H2), jnp.float32),
            jax.ShapeDtypeStruct((H2, DE), jnp.float32),
            jax.ShapeDtypeStruct((H2, DG), jnp.float32),
            jax.ShapeDtypeStruct((H2, 1), jnp.float32),
            jax.ShapeDtypeStruct((H2, 1), jnp.float32),
            jax.ShapeDtypeStruct((2 * OUT_E, H2), jnp.float32),
            jax.ShapeDtypeStruct((2 * OUT_E, 1), jnp.float32),
        ],
    )(pre_ln_g.reshape(1, EDGE_IN), pre_ln_b.reshape(1, EDGE_IN),
      feat_W1, feat_b1.reshape(1, HDDN), feat_W2, feat_b2.reshape(1, OUT_E),
      attn_W1, attn_b1.reshape(1, HDDN), attn_W2, attn_b2.reshape(1, N_HEADS),
      rep)
    Ws, Wr, WeT, WcT, gwc, cvec0, W2BT, b2c = prep_out

    featsT, pooledT = pl.pallas_call(
        _edge_block_kernel,
        grid=(B // BP,),
        in_specs=[
            pl.BlockSpec((BP, N, DN), lambda b: (b, 0, 0)),
            pl.BlockSpec((DE, EW), lambda b: (0, b)),
            pl.BlockSpec((BP, DG, 1), lambda b: (b, 0, 0)),
            full((DN, H2)),
            full((DN, H2)),
            full((H2, DE)),
            full((H2, DG)),
            full((H2, 1)),
            full((H2, 1)),
            full((2 * OUT_E, H2)),
            full((2 * OUT_E, 1)),
            full((2 * N * BP, EW)),
            full((E1, N)),
            full((BP, EW)),
            full((1, DE)),
        ],
        out_specs=[
            pl.BlockSpec((OUT_E, EW), lambda b: (0, b)),
            pl.BlockSpec((BP, OUT_E, N), lambda b: (b, 0, 0)),
        ],
        out_shape=[
            jax.ShapeDtypeStruct((OUT_E, B * E1), jnp.float32),
            jax.ShapeDtypeStruct((B, OUT_E, N), jnp.float32),
        ],
        compiler_params=pltpu.CompilerParams(
            dimension_semantics=("parallel",),
        ),
    )(nodes, edgesT, globsT, Ws, Wr, WeT, WcT, gwc, cvec0, W2BT, b2c,
      selT, selP, ind, onesDE)
    feats = featsT.T
    pooled = pooledT.transpose(0, 2, 1)
    return feats, pooled
